# Initial kernel scaffold; baseline (speedup 1.0000x reference)
import jax
import jax.numpy as jnp
from jax.experimental import pallas as pl


def kernel(pos, h_gcn, table):
    raise NotImplementedError("write your pallas kernel here")


# speedup vs baseline: 7.4505x; 7.4505x over previous
import functools

import jax
import jax.numpy as jnp
from jax import lax
from jax.experimental import pallas as pl
from jax.experimental.pallas import tpu as pltpu
from jax.experimental.pallas import tpu_sc as plsc

D = 128
G = 128
PER_OUT = 256
NC = 2
NS = 16
NW = NC * NS


def _build(b_total):
    assert b_total % (NW * PER_OUT) == 0
    b_per_w = b_total // NW
    gpw = b_per_w // G
    n_iter = b_per_w // PER_OUT

    mesh = plsc.VectorSubcoreMesh(core_axis_name="c", subcore_axis_name="s")

    @functools.partial(
        pl.kernel,
        mesh=mesh,
        out_type=jax.ShapeDtypeStruct((b_total, D), jnp.float32),
        scratch_types=[
            pltpu.VMEM((gpw, G), jnp.int32),
            pltpu.VMEM((2, PER_OUT, D), jnp.float32),
            pltpu.SemaphoreType.DMA,
            pltpu.SemaphoreType.DMA,
        ],
    )
    def gather_kernel(pos_hbm, table_hbm, out_hbm, idx_v, rows_v, sem_g, sem_o):
        wid = lax.axis_index("s") * NC + lax.axis_index("c")
        row_base = wid * b_per_w

        pltpu.sync_copy(pos_hbm.at[pl.ds(wid * gpw, gpw)], idx_v)

        def body(i2, _):
            for p in range(2):
                i = 2 * i2 + p
                off = row_base + i * PER_OUT

                @pl.when(i2 >= 1)
                def _drain(p=p, off=off):
                    pltpu.make_async_copy(
                        rows_v.at[p], out_hbm.at[pl.ds(off, PER_OUT)], sem_o
                    ).wait()

                c0 = pltpu.async_copy(
                    table_hbm.at[idx_v.at[2 * i]],
                    rows_v.at[p, pl.ds(0, G)],
                    sem_g,
                )
                c1 = pltpu.async_copy(
                    table_hbm.at[idx_v.at[2 * i + 1]],
                    rows_v.at[p, pl.ds(G, G)],
                    sem_g,
                )
                c0.wait()
                c1.wait()
                pltpu.async_copy(
                    rows_v.at[p], out_hbm.at[pl.ds(off, PER_OUT)], sem_o
                )
            return ()

        lax.fori_loop(0, n_iter // 2, body, (), unroll=False)

        for p in range(2):
            pltpu.make_async_copy(
                rows_v.at[p], out_hbm.at[pl.ds(row_base, PER_OUT)], sem_o
            ).wait()

    return gather_kernel


@jax.jit
def kernel(pos, h_gcn, table):
    del h_gcn
    b, l, _ = pos.shape
    b_total = b * l
    idx = pos.reshape(b_total // G, G).astype(jnp.int32)
    out = _build(b_total)(idx, table)
    return out.reshape(b, l, D)


# speedup vs baseline: 7.6567x; 1.0277x over previous
import functools

import jax
import jax.numpy as jnp
from jax import lax
from jax.experimental import pallas as pl
from jax.experimental.pallas import tpu as pltpu
from jax.experimental.pallas import tpu_sc as plsc

D = 128
G = 128
NB = 4
NC = 2
NS = 16
NW = NC * NS


def _build(b_total):
    assert b_total % (NW * G * NB) == 0
    b_per_w = b_total // NW
    n_g = b_per_w // G

    mesh = plsc.VectorSubcoreMesh(core_axis_name="c", subcore_axis_name="s")

    @functools.partial(
        pl.kernel,
        mesh=mesh,
        out_type=jax.ShapeDtypeStruct((b_total, D), jnp.float32),
        scratch_types=[
            pltpu.VMEM((n_g, G), jnp.int32),
            pltpu.VMEM((NB, G, D), jnp.float32),
            pltpu.SemaphoreType.DMA,
            pltpu.SemaphoreType.DMA,
        ],
    )
    def gather_kernel(pos_hbm, table_hbm, out_hbm, idx_v, rows_v, sem_g, sem_o):
        wid = lax.axis_index("s") * NC + lax.axis_index("c")
        row_base = wid * b_per_w

        pltpu.sync_copy(pos_hbm.at[pl.ds(wid * n_g, n_g)], idx_v)

        for j in range(2):
            pltpu.async_copy(table_hbm.at[idx_v.at[j]], rows_v.at[j], sem_g)

        def body(i4, _):
            for p in range(NB):
                i = NB * i4 + p

                @pl.when(i >= 2)
                def _drain(p=p, i=i):
                    pltpu.make_async_copy(
                        rows_v.at[(p + 2) % NB],
                        out_hbm.at[pl.ds(row_base + lax.max(i - 2, 0) * G, G)],
                        sem_o,
                    ).wait()

                @pl.when(i + 2 < n_g)
                def _fire(p=p, i=i):
                    pltpu.async_copy(
                        table_hbm.at[idx_v.at[lax.min(i + 2, n_g - 1)]],
                        rows_v.at[(p + 2) % NB],
                        sem_g,
                    )

                pltpu.make_async_copy(
                    table_hbm.at[idx_v.at[i]], rows_v.at[p], sem_g
                ).wait()
                pltpu.async_copy(
                    rows_v.at[p],
                    out_hbm.at[pl.ds(row_base + i * G, G)],
                    sem_o,
                )
            return ()

        lax.fori_loop(0, n_g // NB, body, (), unroll=False)

        for q in range(2):
            i = n_g - 2 + q
            pltpu.make_async_copy(
                rows_v.at[i % NB],
                out_hbm.at[pl.ds(row_base + i * G, G)],
                sem_o,
            ).wait()

    return gather_kernel


@jax.jit
def kernel(pos, h_gcn, table):
    del h_gcn
    b, l, _ = pos.shape
    b_total = b * l
    idx = pos.reshape(b_total // G, G).astype(jnp.int32)
    out = _build(b_total)(idx, table)
    return out.reshape(b, l, D)


# speedup vs baseline: 7.6789x; 1.0029x over previous
import functools

import jax
import jax.numpy as jnp
from jax import lax
from jax.experimental import pallas as pl
from jax.experimental.pallas import tpu as pltpu
from jax.experimental.pallas import tpu_sc as plsc

D = 128
G = 128
NB = 5
LA = 3
NC = 2
NS = 16
NW = NC * NS


def _build(b_total):
    assert (b_total // NW) % (G * NB) == 0
    b_per_w = b_total // NW
    n_g = b_per_w // G

    mesh = plsc.VectorSubcoreMesh(core_axis_name="c", subcore_axis_name="s")

    @functools.partial(
        pl.kernel,
        mesh=mesh,
        out_type=jax.ShapeDtypeStruct((b_total, D), jnp.float32),
        scratch_types=[
            pltpu.VMEM((n_g, G), jnp.int32),
            pltpu.VMEM((NB, G, D), jnp.float32),
            pltpu.SemaphoreType.DMA,
            pltpu.SemaphoreType.DMA,
        ],
    )
    def gather_kernel(pos_hbm, table_hbm, out_hbm, idx_v, rows_v, sem_g, sem_o):
        wid = lax.axis_index("s") * NC + lax.axis_index("c")
        row_base = wid * b_per_w

        pltpu.sync_copy(pos_hbm.at[pl.ds(wid * n_g, n_g)], idx_v)

        for j in range(LA):
            pltpu.async_copy(table_hbm.at[idx_v.at[j]], rows_v.at[j], sem_g)

        def body(i4, _):
            for p in range(NB):
                i = NB * i4 + p

                @pl.when(i >= NB - LA)
                def _drain(p=p, i=i):
                    pltpu.make_async_copy(
                        rows_v.at[(p + LA) % NB],
                        out_hbm.at[
                            pl.ds(row_base + lax.max(i - (NB - LA), 0) * G, G)
                        ],
                        sem_o,
                    ).wait()

                @pl.when(i + LA < n_g)
                def _fire(p=p, i=i):
                    pltpu.async_copy(
                        table_hbm.at[idx_v.at[lax.min(i + LA, n_g - 1)]],
                        rows_v.at[(p + LA) % NB],
                        sem_g,
                    )

                pltpu.make_async_copy(
                    table_hbm.at[idx_v.at[i]], rows_v.at[p], sem_g
                ).wait()
                pltpu.async_copy(
                    rows_v.at[p],
                    out_hbm.at[pl.ds(row_base + i * G, G)],
                    sem_o,
                )
            return ()

        lax.fori_loop(0, n_g // NB, body, (), unroll=False)

        for q in range(NB - LA):
            i = n_g - (NB - LA) + q
            pltpu.make_async_copy(
                rows_v.at[i % NB],
                out_hbm.at[pl.ds(row_base + i * G, G)],
                sem_o,
            ).wait()

    return gather_kernel


@jax.jit
def kernel(pos, h_gcn, table):
    del h_gcn
    b, l, _ = pos.shape
    b_total = b * l
    idx = pos.reshape(b_total // G, G).astype(jnp.int32)
    out = _build(b_total)(idx, table)
    return out.reshape(b, l, D)


# speedup vs baseline: 15.7231x; 2.0476x over previous
import functools

import jax
import jax.numpy as jnp
from jax import lax
from jax.experimental import pallas as pl
from jax.experimental.pallas import tpu as pltpu
from jax.experimental.pallas import tpu_sc as plsc

D = 128
G = 128
NB = 5
LA = 3
NC = 2
NS = 16
NW = NC * NS


def _build(b_total):
    assert (b_total // NW) % (G * NB) == 0
    b_per_w = b_total // NW
    n_g = b_per_w // G

    mesh = plsc.VectorSubcoreMesh(core_axis_name="c", subcore_axis_name="s")

    @functools.partial(
        pl.kernel,
        mesh=mesh,
        out_type=jax.ShapeDtypeStruct((b_total, D), jnp.float32),
        scratch_types=[
            pltpu.VMEM((n_g, G), jnp.int32),
            pltpu.VMEM((NB, G, D), jnp.float32),
            pltpu.VMEM_SHARED((1536, D), jnp.float32),
            pltpu.SemaphoreType.DMA,
            pltpu.SemaphoreType.DMA,
        ],
    )
    def gather_kernel(pos_hbm, table_hbm, out_hbm, idx_v, rows_v, table_sh,
                      sem_g, sem_o):
        sid = lax.axis_index("s")
        wid = sid * NC + lax.axis_index("c")
        row_base = wid * b_per_w

        @pl.when(sid == 0)
        def _stage_table():
            pltpu.sync_copy(table_hbm, table_sh)

        pltpu.sync_copy(pos_hbm.at[pl.ds(wid * n_g, n_g)], idx_v)
        plsc.subcore_barrier()

        for j in range(LA):
            pltpu.async_copy(table_sh.at[idx_v.at[j]], rows_v.at[j], sem_g)

        def body(i4, _):
            for p in range(NB):
                i = NB * i4 + p

                @pl.when(i >= NB - LA)
                def _drain(p=p, i=i):
                    pltpu.make_async_copy(
                        rows_v.at[(p + LA) % NB],
                        out_hbm.at[
                            pl.ds(row_base + lax.max(i - (NB - LA), 0) * G, G)
                        ],
                        sem_o,
                    ).wait()

                @pl.when(i + LA < n_g)
                def _fire(p=p, i=i):
                    pltpu.async_copy(
                        table_sh.at[idx_v.at[lax.min(i + LA, n_g - 1)]],
                        rows_v.at[(p + LA) % NB],
                        sem_g,
                    )

                pltpu.make_async_copy(
                    table_sh.at[idx_v.at[i]], rows_v.at[p], sem_g
                ).wait()
                pltpu.async_copy(
                    rows_v.at[p],
                    out_hbm.at[pl.ds(row_base + i * G, G)],
                    sem_o,
                )
            return ()

        lax.fori_loop(0, n_g // NB, body, (), unroll=False)

        for q in range(NB - LA):
            i = n_g - (NB - LA) + q
            pltpu.make_async_copy(
                rows_v.at[i % NB],
                out_hbm.at[pl.ds(row_base + i * G, G)],
                sem_o,
            ).wait()

    return gather_kernel


@jax.jit
def kernel(pos, h_gcn, table):
    del h_gcn
    b, l, _ = pos.shape
    b_total = b * l
    idx = pos.reshape(b_total // G, G).astype(jnp.int32)
    out = _build(b_total)(idx, table)
    return out.reshape(b, l, D)


# speedup vs baseline: 15.7356x; 1.0008x over previous
import functools

import jax
import jax.numpy as jnp
from jax import lax
from jax.experimental import pallas as pl
from jax.experimental.pallas import tpu as pltpu
from jax.experimental.pallas import tpu_sc as plsc

D = 128
G = 128
NB = 5
LA = 2
NC = 2
NS = 16
NW = NC * NS


def _build(b_total):
    assert (b_total // NW) % (G * NB) == 0
    b_per_w = b_total // NW
    n_g = b_per_w // G

    mesh = plsc.VectorSubcoreMesh(core_axis_name="c", subcore_axis_name="s")

    @functools.partial(
        pl.kernel,
        mesh=mesh,
        out_type=jax.ShapeDtypeStruct((b_total, D), jnp.float32),
        scratch_types=[
            pltpu.VMEM((n_g, G), jnp.int32),
            pltpu.VMEM((NB, G, D), jnp.float32),
            pltpu.VMEM_SHARED((1536, D), jnp.float32),
            pltpu.SemaphoreType.DMA,
            pltpu.SemaphoreType.DMA,
        ],
    )
    def gather_kernel(pos_hbm, table_hbm, out_hbm, idx_v, rows_v, table_sh,
                      sem_g, sem_o):
        sid = lax.axis_index("s")
        wid = sid * NC + lax.axis_index("c")
        row_base = wid * b_per_w

        @pl.when(sid == 0)
        def _stage_table():
            pltpu.sync_copy(table_hbm, table_sh)

        pltpu.sync_copy(pos_hbm.at[pl.ds(wid * n_g, n_g)], idx_v)
        plsc.subcore_barrier()

        for j in range(LA):
            pltpu.async_copy(table_sh.at[idx_v.at[j]], rows_v.at[j], sem_g)

        def body(i4, _):
            for p in range(NB):
                i = NB * i4 + p

                @pl.when(i >= NB - LA)
                def _drain(p=p, i=i):
                    pltpu.make_async_copy(
                        rows_v.at[(p + LA) % NB],
                        out_hbm.at[
                            pl.ds(row_base + lax.max(i - (NB - LA), 0) * G, G)
                        ],
                        sem_o,
                    ).wait()

                @pl.when(i + LA < n_g)
                def _fire(p=p, i=i):
                    pltpu.async_copy(
                        table_sh.at[idx_v.at[lax.min(i + LA, n_g - 1)]],
                        rows_v.at[(p + LA) % NB],
                        sem_g,
                    )

                pltpu.make_async_copy(
                    table_sh.at[idx_v.at[i]], rows_v.at[p], sem_g
                ).wait()
                pltpu.async_copy(
                    rows_v.at[p],
                    out_hbm.at[pl.ds(row_base + i * G, G)],
                    sem_o,
                )
            return ()

        lax.fori_loop(0, n_g // NB, body, (), unroll=False)

        for q in range(NB - LA):
            i = n_g - (NB - LA) + q
            pltpu.make_async_copy(
                rows_v.at[i % NB],
                out_hbm.at[pl.ds(row_base + i * G, G)],
                sem_o,
            ).wait()

    return gather_kernel


@jax.jit
def kernel(pos, h_gcn, table):
    del h_gcn
    b, l, _ = pos.shape
    b_total = b * l
    idx = pos.reshape(b_total // G, G).astype(jnp.int32)
    out = _build(b_total)(idx, table)
    return out.reshape(b, l, D)


# speedup vs baseline: 15.8942x; 1.0101x over previous
import functools

import jax
import jax.numpy as jnp
from jax import lax
from jax.experimental import pallas as pl
from jax.experimental.pallas import tpu as pltpu
from jax.experimental.pallas import tpu_sc as plsc

D = 128
G = 128
NB = 5
LA = 2
NC = 2
NS = 16
NW = NC * NS


def _build(b_total):
    assert (b_total // NW) % (G * NB) == 0
    b_per_w = b_total // NW
    n_g = b_per_w // G

    mesh = plsc.VectorSubcoreMesh(core_axis_name="c", subcore_axis_name="s")

    @functools.partial(
        pl.kernel,
        mesh=mesh,
        out_type=jax.ShapeDtypeStruct((b_total, D), jnp.float32),
        scratch_types=[
            pltpu.VMEM((n_g, G), jnp.int32),
            pltpu.VMEM((NB, G, D), jnp.float32),
            pltpu.VMEM_SHARED((1536, D), jnp.float32),
            pltpu.SemaphoreType.DMA,
            pltpu.SemaphoreType.DMA,
            pltpu.SemaphoreType.DMA,
        ],
    )
    def gather_kernel(pos_hbm, table_hbm, out_hbm, idx_v, rows_v, table_sh,
                      sem_g, sem_h, sem_o):
        sid = lax.axis_index("s")
        wid = sid * NC + lax.axis_index("c")
        row_base = wid * b_per_w

        @pl.when(sid == 0)
        def _stage_table():
            pltpu.sync_copy(table_hbm, table_sh)

        pltpu.sync_copy(pos_hbm.at[pl.ds(wid * n_g, n_g)], idx_v)
        plsc.subcore_barrier()

        pltpu.async_copy(table_hbm.at[idx_v.at[0]], rows_v.at[0], sem_h)
        for j in range(1, LA):
            pltpu.async_copy(table_sh.at[idx_v.at[j]], rows_v.at[j], sem_g)

        def body(i4, _):
            for p in range(NB):
                i = NB * i4 + p

                @pl.when(i >= NB - LA)
                def _drain(p=p, i=i):
                    pltpu.make_async_copy(
                        rows_v.at[(p + LA) % NB],
                        out_hbm.at[
                            pl.ds(row_base + lax.max(i - (NB - LA), 0) * G, G)
                        ],
                        sem_o,
                    ).wait()

                @pl.when(i + LA < n_g)
                def _fire(p=p, i=i, i4=i4):
                    j_clamped = lax.min(i + LA, n_g - 1)
                    dst = rows_v.at[(p + LA) % NB]
                    if p == 3:
                        hbm_turn = lax.rem(i4, 4) == 3

                        @pl.when(hbm_turn)
                        def _from_hbm():
                            pltpu.async_copy(
                                table_hbm.at[idx_v.at[j_clamped]], dst, sem_h
                            )

                        @pl.when(jnp.logical_not(hbm_turn))
                        def _from_spmem():
                            pltpu.async_copy(
                                table_sh.at[idx_v.at[j_clamped]], dst, sem_g
                            )
                    else:
                        pltpu.async_copy(
                            table_sh.at[idx_v.at[j_clamped]], dst, sem_g
                        )

                if p == 0:
                    hbm_wait = lax.rem(i4, 4) == 0

                    @pl.when(hbm_wait)
                    def _wait_hbm(p=p, i=i):
                        pltpu.make_async_copy(
                            table_hbm.at[idx_v.at[i]], rows_v.at[p], sem_h
                        ).wait()

                    @pl.when(jnp.logical_not(hbm_wait))
                    def _wait_spmem(p=p, i=i):
                        pltpu.make_async_copy(
                            table_sh.at[idx_v.at[i]], rows_v.at[p], sem_g
                        ).wait()
                else:
                    pltpu.make_async_copy(
                        table_sh.at[idx_v.at[i]], rows_v.at[p], sem_g
                    ).wait()
                pltpu.async_copy(
                    rows_v.at[p],
                    out_hbm.at[pl.ds(row_base + i * G, G)],
                    sem_o,
                )
            return ()

        lax.fori_loop(0, n_g // NB, body, (), unroll=False)

        for q in range(NB - LA):
            i = n_g - (NB - LA) + q
            pltpu.make_async_copy(
                rows_v.at[i % NB],
                out_hbm.at[pl.ds(row_base + i * G, G)],
                sem_o,
            ).wait()

    return gather_kernel


@jax.jit
def kernel(pos, h_gcn, table):
    del h_gcn
    b, l, _ = pos.shape
    b_total = b * l
    idx = pos.reshape(b_total // G, G).astype(jnp.int32)
    out = _build(b_total)(idx, table)
    return out.reshape(b, l, D)


# speedup vs baseline: 15.9295x; 1.0022x over previous
import functools

import jax
import jax.numpy as jnp
from jax import lax
from jax.experimental import pallas as pl
from jax.experimental.pallas import tpu as pltpu
from jax.experimental.pallas import tpu_sc as plsc

D = 128
G = 128
NB = 5
LA = 2
NC = 2
NS = 16
NW = NC * NS


def _build(b_total):
    assert (b_total // NW) % (G * NB) == 0
    b_per_w = b_total // NW
    n_g = b_per_w // G

    mesh = plsc.VectorSubcoreMesh(core_axis_name="c", subcore_axis_name="s")

    @functools.partial(
        pl.kernel,
        mesh=mesh,
        out_type=jax.ShapeDtypeStruct((b_total, D), jnp.float32),
        scratch_types=[
            pltpu.VMEM((n_g, G), jnp.int32),
            pltpu.VMEM((NB, G, D), jnp.float32),
            pltpu.VMEM_SHARED((1536, D), jnp.float32),
            pltpu.SemaphoreType.DMA,
            pltpu.SemaphoreType.DMA,
            pltpu.SemaphoreType.DMA,
        ],
    )
    def gather_kernel(pos_hbm, table_hbm, out_hbm, idx_v, rows_v, table_sh,
                      sem_g, sem_h, sem_o):
        sid = lax.axis_index("s")
        wid = sid * NC + lax.axis_index("c")
        row_base = wid * b_per_w

        rows_per_tile = 1536 // NS
        pltpu.sync_copy(
            table_hbm.at[pl.ds(sid * rows_per_tile, rows_per_tile)],
            table_sh.at[pl.ds(sid * rows_per_tile, rows_per_tile)],
        )

        pltpu.sync_copy(pos_hbm.at[pl.ds(wid * n_g, n_g)], idx_v)
        plsc.subcore_barrier()

        pltpu.async_copy(table_hbm.at[idx_v.at[0]], rows_v.at[0], sem_h)
        for j in range(1, LA):
            pltpu.async_copy(table_sh.at[idx_v.at[j]], rows_v.at[j], sem_g)

        def body(i4, _):
            for p in range(NB):
                i = NB * i4 + p

                @pl.when(i >= NB - LA)
                def _drain(p=p, i=i):
                    pltpu.make_async_copy(
                        rows_v.at[(p + LA) % NB],
                        out_hbm.at[
                            pl.ds(row_base + lax.max(i - (NB - LA), 0) * G, G)
                        ],
                        sem_o,
                    ).wait()

                @pl.when(i + LA < n_g)
                def _fire(p=p, i=i, i4=i4):
                    j_clamped = lax.min(i + LA, n_g - 1)
                    dst = rows_v.at[(p + LA) % NB]
                    if p == 3:
                        hbm_turn = lax.rem(i4, 4) == 3

                        @pl.when(hbm_turn)
                        def _from_hbm():
                            pltpu.async_copy(
                                table_hbm.at[idx_v.at[j_clamped]], dst, sem_h
                            )

                        @pl.when(jnp.logical_not(hbm_turn))
                        def _from_spmem():
                            pltpu.async_copy(
                                table_sh.at[idx_v.at[j_clamped]], dst, sem_g
                            )
                    else:
                        pltpu.async_copy(
                            table_sh.at[idx_v.at[j_clamped]], dst, sem_g
                        )

                if p == 0:
                    hbm_wait = lax.rem(i4, 4) == 0

                    @pl.when(hbm_wait)
                    def _wait_hbm(p=p, i=i):
                        pltpu.make_async_copy(
                            table_hbm.at[idx_v.at[i]], rows_v.at[p], sem_h
                        ).wait()

                    @pl.when(jnp.logical_not(hbm_wait))
                    def _wait_spmem(p=p, i=i):
                        pltpu.make_async_copy(
                            table_sh.at[idx_v.at[i]], rows_v.at[p], sem_g
                        ).wait()
                else:
                    pltpu.make_async_copy(
                        table_sh.at[idx_v.at[i]], rows_v.at[p], sem_g
                    ).wait()
                pltpu.async_copy(
                    rows_v.at[p],
                    out_hbm.at[pl.ds(row_base + i * G, G)],
                    sem_o,
                )
            return ()

        lax.fori_loop(0, n_g // NB, body, (), unroll=False)

        for q in range(NB - LA):
            i = n_g - (NB - LA) + q
            pltpu.make_async_copy(
                rows_v.at[i % NB],
                out_hbm.at[pl.ds(row_base + i * G, G)],
                sem_o,
            ).wait()

    return gather_kernel


@jax.jit
def kernel(pos, h_gcn, table):
    del h_gcn
    b, l, _ = pos.shape
    b_total = b * l
    idx = pos.reshape(b_total // G, G).astype(jnp.int32)
    out = _build(b_total)(idx, table)
    return out.reshape(b, l, D)
